# Initial kernel scaffold; baseline (speedup 1.0000x reference)
import functools

import jax
import jax.numpy as jnp
from jax import lax
from jax.experimental import pallas as pl
from jax.experimental.pallas import tpu as pltpu
from jax.experimental.pallas import tpu_sc as plsc

N = 10000
E = 320000
D = 128
T = 12

NC = 2
NS = 16
NW = NC * NS

EB = 80
EPT_DEG = E // NW
EPT = E // NS
BATCHES = EPT // EB
RPT = N // NS
CPC = T // NC

BN = 400
GRID = N // BN


def _deg_body(dst_hbm, out_hbm, acc_v, idx_v):
    c = lax.axis_index("c")
    s = lax.axis_index("s")
    wid = c * NS + s
    zero16 = jnp.zeros((16,), jnp.float32)
    ones16 = jnp.ones((16,), jnp.float32)

    def zero_step(i, carry):
        acc_v[pl.ds(i * 16, 16)] = zero16
        return carry

    lax.fori_loop(0, N // 16, zero_step, 0)

    pltpu.sync_copy(dst_hbm.at[pl.ds(wid * EPT_DEG, EPT_DEG)], idx_v)

    def step(i, carry):
        idx = idx_v[pl.ds(i * 16, 16)]
        plsc.addupdate_scatter(acc_v, [idx], ones16)
        return carry

    lax.fori_loop(0, EPT_DEG // 16, step, 0)
    pltpu.sync_copy(acc_v, out_hbm.at[wid])


def _deg_partials(dst):
    return pl.kernel(
        _deg_body,
        out_type=jax.ShapeDtypeStruct((NW, N), jnp.float32),
        mesh=plsc.VectorSubcoreMesh(
            core_axis_name="c", subcore_axis_name="s",
            num_cores=NC, num_subcores=NS),
        scratch_types=[
            pltpu.VMEM((N,), jnp.float32),
            pltpu.VMEM((EPT_DEG,), jnp.int32),
        ],
    )(dst)


def _prescale_body(xt_ref, degp_ref, xp_ref):
    deg = jnp.sum(degp_ref[...], axis=1, keepdims=True) + 1.0
    dinv = lax.rsqrt(deg)
    xp_ref[...] = xt_ref[...] * dinv[None, :, :]


def _prescale(xt, degt):
    return pl.pallas_call(
        _prescale_body,
        grid=(GRID,),
        in_specs=[
            pl.BlockSpec((T, BN, D), lambda i: (0, i, 0)),
            pl.BlockSpec((BN, NW), lambda i: (i, 0)),
        ],
        out_specs=pl.BlockSpec((T, BN, D), lambda i: (0, i, 0)),
        out_shape=jax.ShapeDtypeStruct((T, N, D), jnp.float32),
    )(xt, degt)


def _spmm_body(xp_hbm, src2d_hbm, dst2d_hbm, out_hbm, acc_sh,
               rows0, rows1, src0, src1, dst0, dst1, zbuf, sem0, sem1):
    c = lax.axis_index("c")
    s = lax.axis_index("s")
    zero16 = jnp.zeros((16,), jnp.float32)

    def zfill(i, carry):
        for l in range(D // 16):
            zbuf[i, pl.ds(l * 16, 16)] = zero16
        return carry

    lax.fori_loop(0, 125, zfill, 0)

    bufs = ((src0, dst0, rows0, sem0), (src1, dst1, rows1, sem1))
    base_row = s * BATCHES

    def chunk_step(j, carry):
        t = c * CPC + j
        toff = t * N

        for z in range(RPT // 125):
            pltpu.sync_copy(zbuf, acc_sh.at[pl.ds(s * RPT + z * 125, 125)])
        plsc.subcore_barrier()

        def stage(par, row):
            srcv, dstv, rows, sem = bufs[par]
            pltpu.sync_copy(src2d_hbm.at[row], srcv)
            pltpu.sync_copy(dst2d_hbm.at[row], dstv)
            for k in range(EB // 16):
                srcv[pl.ds(k * 16, 16)] = srcv[pl.ds(k * 16, 16)] + toff
            pltpu.async_copy(xp_hbm.at[srcv], rows, sem)

        def consume(par):
            srcv, dstv, rows, sem = bufs[par]
            pltpu.make_async_copy(xp_hbm.at[srcv], rows, sem).wait()
            pltpu.sync_copy(rows, acc_sh.at[dstv], add=True)

        stage(0, base_row)

        def pair_step(i, carry2):
            r = base_row + 2 * i
            stage(1, r + 1)
            consume(0)

            @pl.when(i < BATCHES // 2 - 1)
            def _():
                stage(0, r + 2)

            consume(1)
            return carry2

        lax.fori_loop(0, BATCHES // 2, pair_step, 0)
        plsc.subcore_barrier()
        pltpu.sync_copy(
            acc_sh.at[pl.ds(s * RPT, RPT)],
            out_hbm.at[pl.ds(t * N + s * RPT, RPT)])
        return carry

    lax.fori_loop(0, CPC, chunk_step, 0)


def _spmm(xp_flat, src2d, dst2d):
    return pl.kernel(
        _spmm_body,
        out_type=jax.ShapeDtypeStruct((T * N, D), jnp.float32),
        mesh=plsc.VectorSubcoreMesh(
            core_axis_name="c", subcore_axis_name="s",
            num_cores=NC, num_subcores=NS),
        scratch_types=[
            pltpu.VMEM_SHARED((N, D), jnp.float32),
            pltpu.VMEM((EB, D), jnp.float32),
            pltpu.VMEM((EB, D), jnp.float32),
            pltpu.VMEM((EB,), jnp.int32),
            pltpu.VMEM((EB,), jnp.int32),
            pltpu.VMEM((EB,), jnp.int32),
            pltpu.VMEM((EB,), jnp.int32),
            pltpu.VMEM((125, D), jnp.float32),
            pltpu.SemaphoreType.DMA,
            pltpu.SemaphoreType.DMA,
        ],
    )(xp_flat, src2d, dst2d)


def _dense_body(s_ref, xp_ref, degp_ref, wz_ref, lz_ref, wh_ref, lh_ref,
                bz_ref, lzb_ref, bh_ref, lhb_ref, att_ref, wo_ref, bo_ref,
                out_ref):
    deg = jnp.sum(degp_ref[...], axis=1, keepdims=True) + 1.0
    dinv = lax.rsqrt(deg)

    att = att_ref[...]
    m = jnp.max(att, axis=1, keepdims=True)
    ea = jnp.exp(att - m)
    p = ea / jnp.sum(ea, axis=1, keepdims=True)

    wz = wz_ref[...]
    lz = lz_ref[...]
    wh = wh_ref[...]
    lh = lh_ref[...]
    bz = bz_ref[...]
    lzb = lzb_ref[...]
    bh = bh_ref[...]
    lhb = lhb_ref[...]

    acc = jnp.zeros((BN, D), jnp.float32)
    for t in range(T):
        ax = dinv * (s_ref[t] + xp_ref[t])
        gz = jnp.dot(ax, wz, preferred_element_type=jnp.float32) + bz
        gz = jnp.dot(gz, lz, preferred_element_type=jnp.float32) + lzb
        gh = jnp.dot(ax, wh, preferred_element_type=jnp.float32) + bh
        gh = jnp.dot(gh, lh, preferred_element_type=jnp.float32) + lhb
        h = (1.0 - jax.nn.sigmoid(gz)) * jnp.tanh(gh)
        acc = acc + p[0, t] * h

    out_ref[...] = (jnp.dot(jax.nn.relu(acc), wo_ref[...],
                            preferred_element_type=jnp.float32) + bo_ref[...])


def _dense(s3, xp, degt, wz, lz, wh, lh, bz, lzb, bh, lhb, att2, wo, bo):
    def full(shape):
        nd = len(shape)
        return pl.BlockSpec(shape, lambda i, _nd=nd: (0,) * _nd)
    return pl.pallas_call(
        _dense_body,
        grid=(GRID,),
        in_specs=[
            pl.BlockSpec((T, BN, D), lambda i: (0, i, 0)),
            pl.BlockSpec((T, BN, D), lambda i: (0, i, 0)),
            pl.BlockSpec((BN, NW), lambda i: (i, 0)),
            full((D, D)), full((D, D)), full((D, D)), full((D, D)),
            full((1, D)), full((1, D)), full((1, D)), full((1, D)),
            full((1, T)), full((D, T)), full((1, T)),
        ],
        out_specs=pl.BlockSpec((BN, T), lambda i: (i, 0)),
        out_shape=jax.ShapeDtypeStruct((N, T), jnp.float32),
    )(s3, xp, degt, wz, lz, wh, lh, bz, lzb, bh, lhb, att2, wo, bo)


def kernel(x_1, edge_index_1, x_2, edge_index_2, W_z, b_z, W_r, b_r, W_h, b_h,
           lin_z_W, lin_z_b, lin_r_W, lin_r_b, lin_h_W, lin_h_b, att, W_out,
           b_out):
    src = edge_index_1[0]
    dst = edge_index_1[1]

    xt = jnp.transpose(x_1, (2, 0, 1))
    src2d = src.reshape(E // EB, EB)
    dst2d = dst.reshape(E // EB, EB)

    degp = _deg_partials(dst)
    degt = jnp.transpose(degp)

    xp = _prescale(xt, degt)
    s_flat = _spmm(xp.reshape(T * N, D), src2d, dst2d)
    s3 = s_flat.reshape(T, N, D)

    return _dense(
        s3, xp, degt,
        W_z, lin_z_W[:D], W_h, lin_h_W[:D],
        b_z.reshape(1, D), lin_z_b.reshape(1, D),
        b_h.reshape(1, D), lin_h_b.reshape(1, D),
        att.reshape(1, T), W_out, b_out.reshape(1, T))


# speedup vs baseline: 30.9662x; 30.9662x over previous
import functools

import jax
import jax.numpy as jnp
from jax import lax
from jax.experimental import pallas as pl
from jax.experimental.pallas import tpu as pltpu
from jax.experimental.pallas import tpu_sc as plsc

N = 10000
E = 320000
D = 128
T = 12

NC = 2
NS = 16
NW = NC * NS

EB = 80
EPT_DEG = E // NW
EPT = E // NS
BATCHES = EPT // EB
RPT = N // NS
CPC = T // NC

BN = 400
GRID = N // BN


def _deg_body(dst_hbm, out_hbm, acc_v, idx_v):
    c = lax.axis_index("c")
    s = lax.axis_index("s")
    wid = c * NS + s
    zero16 = jnp.zeros((16,), jnp.float32)
    ones16 = jnp.ones((16,), jnp.float32)

    def zero_step(i, carry):
        acc_v[pl.ds(i * 16, 16)] = zero16
        return carry

    lax.fori_loop(0, N // 16, zero_step, 0)

    pltpu.sync_copy(dst_hbm.at[pl.ds(wid * EPT_DEG, EPT_DEG)], idx_v)

    def step(i, carry):
        idx = idx_v[pl.ds(i * 16, 16)]
        plsc.addupdate_scatter(acc_v, [idx], ones16)
        return carry

    lax.fori_loop(0, EPT_DEG // 16, step, 0)
    pltpu.sync_copy(acc_v, out_hbm.at[wid])


def _deg_partials(dst):
    return pl.kernel(
        _deg_body,
        out_type=jax.ShapeDtypeStruct((NW, N), jnp.float32),
        mesh=plsc.VectorSubcoreMesh(
            core_axis_name="c", subcore_axis_name="s",
            num_cores=NC, num_subcores=NS),
        scratch_types=[
            pltpu.VMEM((N,), jnp.float32),
            pltpu.VMEM((EPT_DEG,), jnp.int32),
        ],
        compiler_params=pltpu.CompilerParams(
            needs_layout_passes=False, use_tc_tiling_on_sc=False),
    )(dst)


def _prescale_body(xt_ref, degp_ref, xp_ref):
    deg = jnp.sum(degp_ref[...], axis=1, keepdims=True) + 1.0
    dinv = lax.rsqrt(deg)
    xp_ref[...] = xt_ref[...] * dinv[None, :, :]


def _prescale(xt, degt):
    return pl.pallas_call(
        _prescale_body,
        grid=(GRID,),
        in_specs=[
            pl.BlockSpec((T, BN, D), lambda i: (0, i, 0)),
            pl.BlockSpec((BN, NW), lambda i: (i, 0)),
        ],
        out_specs=pl.BlockSpec((T, BN, D), lambda i: (0, i, 0)),
        out_shape=jax.ShapeDtypeStruct((T, N, D), jnp.float32),
    )(xt, degt)


def _spmm_body(xp_hbm, src2d_hbm, dst2d_hbm, out_hbm, acc_sh,
               rows0, rows1, src0, src1, dst0, dst1, zbuf, sem0, sem1):
    c = lax.axis_index("c")
    s = lax.axis_index("s")
    zero16 = jnp.zeros((16,), jnp.float32)

    def zfill(i, carry):
        for l in range(D // 16):
            zbuf[i, pl.ds(l * 16, 16)] = zero16
        return carry

    lax.fori_loop(0, 125, zfill, 0)

    bufs = ((src0, dst0, rows0, sem0), (src1, dst1, rows1, sem1))
    base_row = s * BATCHES

    def chunk_step(j, carry):
        t = c * CPC + j
        toff = t * N

        for z in range(RPT // 125):
            pltpu.sync_copy(zbuf, acc_sh.at[pl.ds(s * RPT + z * 125, 125)])
        plsc.subcore_barrier()

        def stage(par, row):
            srcv, dstv, rows, sem = bufs[par]
            pltpu.sync_copy(src2d_hbm.at[row], srcv)
            pltpu.sync_copy(dst2d_hbm.at[row], dstv)
            for k in range(EB // 16):
                srcv[pl.ds(k * 16, 16)] = srcv[pl.ds(k * 16, 16)] + toff
            pltpu.async_copy(xp_hbm.at[srcv], rows, sem)

        def consume(par):
            srcv, dstv, rows, sem = bufs[par]
            pltpu.make_async_copy(xp_hbm.at[srcv], rows, sem).wait()
            pltpu.sync_copy(rows, acc_sh.at[dstv], add=True)

        stage(0, base_row)

        def pair_step(i, carry2):
            r = base_row + 2 * i
            stage(1, r + 1)
            consume(0)

            @pl.when(i < BATCHES // 2 - 1)
            def _():
                stage(0, r + 2)

            consume(1)
            return carry2

        lax.fori_loop(0, BATCHES // 2, pair_step, 0)
        plsc.subcore_barrier()
        pltpu.sync_copy(
            acc_sh.at[pl.ds(s * RPT, RPT)],
            out_hbm.at[pl.ds(t * N + s * RPT, RPT)])
        return carry

    lax.fori_loop(0, CPC, chunk_step, 0)


def _spmm(xp_flat, src2d, dst2d):
    return pl.kernel(
        _spmm_body,
        out_type=jax.ShapeDtypeStruct((T * N, D), jnp.float32),
        mesh=plsc.VectorSubcoreMesh(
            core_axis_name="c", subcore_axis_name="s",
            num_cores=NC, num_subcores=NS),
        scratch_types=[
            pltpu.VMEM_SHARED((N, D), jnp.float32),
            pltpu.VMEM((EB, D), jnp.float32),
            pltpu.VMEM((EB, D), jnp.float32),
            pltpu.VMEM((EB,), jnp.int32),
            pltpu.VMEM((EB,), jnp.int32),
            pltpu.VMEM((EB,), jnp.int32),
            pltpu.VMEM((EB,), jnp.int32),
            pltpu.VMEM((125, D), jnp.float32),
            pltpu.SemaphoreType.DMA,
            pltpu.SemaphoreType.DMA,
        ],
        compiler_params=pltpu.CompilerParams(
            needs_layout_passes=False, use_tc_tiling_on_sc=False),
    )(xp_flat, src2d, dst2d)


def _dense_body(s_ref, xp_ref, degp_ref, wz_ref, lz_ref, wh_ref, lh_ref,
                bz_ref, lzb_ref, bh_ref, lhb_ref, att_ref, wo_ref, bo_ref,
                out_ref):
    deg = jnp.sum(degp_ref[...], axis=1, keepdims=True) + 1.0
    dinv = lax.rsqrt(deg)

    att = att_ref[...]
    m = jnp.max(att, axis=1, keepdims=True)
    ea = jnp.exp(att - m)
    p = ea / jnp.sum(ea, axis=1, keepdims=True)

    wz = wz_ref[...]
    lz = lz_ref[...]
    wh = wh_ref[...]
    lh = lh_ref[...]
    bz = bz_ref[...]
    lzb = lzb_ref[...]
    bh = bh_ref[...]
    lhb = lhb_ref[...]

    acc = jnp.zeros((BN, D), jnp.float32)
    for t in range(T):
        ax = dinv * (s_ref[t] + xp_ref[t])
        gz = jnp.dot(ax, wz, preferred_element_type=jnp.float32) + bz
        gz = jnp.dot(gz, lz, preferred_element_type=jnp.float32) + lzb
        gh = jnp.dot(ax, wh, preferred_element_type=jnp.float32) + bh
        gh = jnp.dot(gh, lh, preferred_element_type=jnp.float32) + lhb
        h = (1.0 - jax.nn.sigmoid(gz)) * jnp.tanh(gh)
        acc = acc + p[0, t] * h

    out_ref[...] = (jnp.dot(jax.nn.relu(acc), wo_ref[...],
                            preferred_element_type=jnp.float32) + bo_ref[...])


def _dense(s3, xp, degt, wz, lz, wh, lh, bz, lzb, bh, lhb, att2, wo, bo):
    def full(shape):
        nd = len(shape)
        return pl.BlockSpec(shape, lambda i, _nd=nd: (0,) * _nd)
    return pl.pallas_call(
        _dense_body,
        grid=(GRID,),
        in_specs=[
            pl.BlockSpec((T, BN, D), lambda i: (0, i, 0)),
            pl.BlockSpec((T, BN, D), lambda i: (0, i, 0)),
            pl.BlockSpec((BN, NW), lambda i: (i, 0)),
            full((D, D)), full((D, D)), full((D, D)), full((D, D)),
            full((1, D)), full((1, D)), full((1, D)), full((1, D)),
            full((1, T)), full((D, T)), full((1, T)),
        ],
        out_specs=pl.BlockSpec((BN, T), lambda i: (i, 0)),
        out_shape=jax.ShapeDtypeStruct((N, T), jnp.float32),
    )(s3, xp, degt, wz, lz, wh, lh, bz, lzb, bh, lhb, att2, wo, bo)


def kernel(x_1, edge_index_1, x_2, edge_index_2, W_z, b_z, W_r, b_r, W_h, b_h,
           lin_z_W, lin_z_b, lin_r_W, lin_r_b, lin_h_W, lin_h_b, att, W_out,
           b_out):
    src = edge_index_1[0]
    dst = edge_index_1[1]

    xt = jnp.transpose(x_1, (2, 0, 1))
    src2d = src.reshape(E // EB, EB)
    dst2d = dst.reshape(E // EB, EB)

    degp = _deg_partials(dst)
    degt = jnp.transpose(degp)

    xp = _prescale(xt, degt)
    s_flat = _spmm(xp.reshape(T * N, D), src2d, dst2d)
    s3 = s_flat.reshape(T, N, D)

    return _dense(
        s3, xp, degt,
        W_z, lin_z_W[:D], W_h, lin_h_W[:D],
        b_z.reshape(1, D), lin_z_b.reshape(1, D),
        b_h.reshape(1, D), lin_h_b.reshape(1, D),
        att.reshape(1, T), W_out, b_out.reshape(1, T))
